# Initial kernel scaffold; baseline (speedup 1.0000x reference)
import jax
import jax.numpy as jnp
from jax.experimental import pallas as pl


def kernel(hidden_states, router_weight, merged_gate_up_proj, merged_down_proj):
    raise NotImplementedError("write your pallas kernel here")


# speedup vs baseline: 1.9172x; 1.9172x over previous
import functools

import jax
import jax.numpy as jnp
from jax.experimental import pallas as pl
from jax.experimental.pallas import tpu as pltpu

T = 4096
D = 1024
F = 512
E = 8
K = 2

BT = 512


def _moe_dense_kernel(hs_f32_ref, hs_bf_ref, rw_ref, wgu_ref, wd_ref,
                      out_ref, gates_ref):
    j = pl.program_id(1)

    @pl.when(j == 0)
    def _router():
        x = hs_f32_ref[...]
        logits = jax.lax.dot_general(
            x, rw_ref[...], (((1,), (1,)), ((), ())),
            preferred_element_type=jnp.float32)
        m = jnp.max(logits, axis=-1, keepdims=True)
        ex = jnp.exp(logits - m)
        probs = ex / jnp.sum(ex, axis=-1, keepdims=True)
        cols = jax.lax.broadcasted_iota(jnp.int32, (BT, E), 1)
        i1 = jnp.argmax(probs, axis=-1, keepdims=True)
        is1 = cols == i1
        probs_m = jnp.where(is1, -1.0, probs)
        i2 = jnp.argmax(probs_m, axis=-1, keepdims=True)
        is2 = cols == i2
        gates_ref[...] = jnp.where(is1 | is2, probs, 0.0)

    x_bf = hs_bf_ref[...]
    gu = jnp.dot(x_bf, wgu_ref[0], preferred_element_type=jnp.float32)
    g = gu[:, :F]
    u = gu[:, F:]
    h = (g * jax.lax.logistic(g) * u).astype(jnp.bfloat16)
    y = jnp.dot(h, wd_ref[0], preferred_element_type=jnp.float32)
    gates = gates_ref[...]
    ecols = jax.lax.broadcasted_iota(jnp.int32, (BT, E), 1)
    gate_j = jnp.sum(jnp.where(ecols == j, gates, 0.0), axis=-1,
                     keepdims=True)
    contrib = gate_j * y

    @pl.when(j == 0)
    def _init():
        out_ref[...] = contrib

    @pl.when(j > 0)
    def _acc():
        out_ref[...] += contrib


@jax.jit
def kernel(hidden_states, router_weight, merged_gate_up_proj, merged_down_proj):
    hs_bf = hidden_states.astype(jnp.bfloat16)
    wgu_bf = merged_gate_up_proj.astype(jnp.bfloat16)
    wd_bf = merged_down_proj.astype(jnp.bfloat16)

    grid = (T // BT, E)
    out = pl.pallas_call(
        _moe_dense_kernel,
        grid=grid,
        in_specs=[
            pl.BlockSpec((BT, D), lambda i, j: (i, 0)),
            pl.BlockSpec((BT, D), lambda i, j: (i, 0)),
            pl.BlockSpec((E, D), lambda i, j: (0, 0)),
            pl.BlockSpec((1, D, 2 * F), lambda i, j: (j, 0, 0)),
            pl.BlockSpec((1, F, D), lambda i, j: (j, 0, 0)),
        ],
        out_specs=pl.BlockSpec((BT, D), lambda i, j: (i, 0)),
        out_shape=jax.ShapeDtypeStruct((T, D), jnp.float32),
        scratch_shapes=[pltpu.VMEM((BT, E), jnp.float32)],
        compiler_params=pltpu.CompilerParams(
            dimension_semantics=("arbitrary", "arbitrary"),
        ),
    )(hidden_states, hs_bf, router_weight, wgu_bf, wd_bf)
    return out
